# baseline (speedup 1.0000x reference)
import functools

import jax
import jax.numpy as jnp
from jax import lax
from jax.experimental import pallas as pl
from jax.experimental.pallas import tpu as pltpu
from jax.experimental.pallas import tpu_sc as plsc

K = 256
L = 16

_ONES = None


def _clear_hist(hist_ref, nwords):
    def body(i, _):
        hist_ref[pl.ds(i * L, L)] = jnp.zeros((L,), jnp.int32)
        return 0

    lax.fori_loop(0, nwords // L, body, 0, unroll=8)


def _merge_hist(hist_ref, hist2_ref):
    zeros = jnp.zeros((L,), jnp.int32)

    def body(jb, _):
        acc = zeros
        for lane in range(L):
            off = lane * 256 + jb * L
            acc = acc + hist_ref[pl.ds(off, L)]
            hist_ref[pl.ds(off, L)] = zeros
        hist2_ref[pl.ds(jb * L, L)] = acc
        return 0

    lax.fori_loop(0, 256 // L, body, 0)


def _scan_level(hist_ref, r):
    lanes = lax.iota(jnp.int32, L)

    def body(jj, carry):
        suffix, best_b, cge_b, h_b = carry
        j = 15 - jj
        v = hist_ref[pl.ds(j * L, L)]
        rv = lax.rev(v, (0,))
        cs = plsc.cumsum(rv) + suffix
        m = cs >= r
        npos = jnp.max(plsc.all_reduce_population_count(m))
        l = jnp.max(plsc.all_reduce_ffs(m))
        sel = lanes == l
        cs_l = jnp.max(jnp.where(sel, cs, 0))
        rv_l = jnp.max(jnp.where(sel, rv, 0))
        b = j * L + (L - 1) - l
        upd = jnp.logical_and(npos > 0, best_b < 0)
        best_b = jnp.where(upd, b, best_b)
        cge_b = jnp.where(upd, cs_l, cge_b)
        h_b = jnp.where(upd, rv_l, h_b)
        suffix = suffix + jnp.sum(v)
        return suffix, best_b, cge_b, h_b

    z32 = jnp.int32(0)
    _, best_b, cge_b, h_b = lax.fori_loop(
        0, 256 // L, body, (z32, jnp.int32(-1), z32, z32)
    )
    return best_b, r - (cge_b - h_b)


def _row_select(row_ref, hist_ref, hist2_ref, nvec):
    ones = jnp.ones((L,), jnp.int32)
    lane_off = lax.iota(jnp.int32, L) * 256

    def l0(i, _):
        v = row_ref[pl.ds(i * L, L)]
        z = jnp.maximum(lax.bitcast_convert_type(v, jnp.int32), 0)
        row_ref[pl.ds(i * L, L)] = lax.bitcast_convert_type(z, jnp.float32)
        idx = lane_off + lax.shift_right_logical(z, 24)
        plsc.addupdate_scatter(hist_ref, [idx], ones)
        return 0

    lax.fori_loop(0, nvec, l0, 0, unroll=8)
    _merge_hist(hist_ref, hist2_ref)
    b, r = _scan_level(hist2_ref, jnp.int32(K))
    prefix = b << 24

    for shift in (16, 8, 0):
        ph = lax.shift_right_logical(prefix, shift + 8)

        def lj(i, _, shift=shift, ph=ph):
            z = lax.bitcast_convert_type(row_ref[pl.ds(i * L, L)], jnp.int32)
            m = lax.shift_right_logical(z, shift + 8) == ph
            idx = lane_off + (lax.shift_right_logical(z, shift) & 255)
            plsc.addupdate_scatter(hist_ref, [idx], ones, mask=m)
            return 0

        lax.fori_loop(0, nvec, lj, 0, unroll=8)
        _merge_hist(hist_ref, hist2_ref)
        b, r = _scan_level(hist2_ref, r)
        prefix = prefix | (b << shift)

    def out_body(i, _):
        zf = row_ref[pl.ds(i * L, L)]
        z = lax.bitcast_convert_type(zf, jnp.int32)
        row_ref[pl.ds(i * L, L)] = jnp.where(z >= prefix, zf, 0.0)
        return 0

    lax.fori_loop(0, nvec, out_body, 0, unroll=8)


def kernel(x):
    m, n = x.shape
    nw = 32
    rows_per_w = m // nw
    nvec = n // L
    mesh = plsc.VectorSubcoreMesh(
        core_axis_name="c", subcore_axis_name="s", num_cores=2, num_subcores=16
    )

    @functools.partial(
        pl.kernel,
        out_type=jax.ShapeDtypeStruct((m, n), jnp.float32),
        mesh=mesh,
        scratch_types=[
            pltpu.VMEM((n,), jnp.float32),
            pltpu.VMEM((256 * L,), jnp.int32),
            pltpu.VMEM((256,), jnp.int32),
        ],
        compiler_params=pltpu.CompilerParams(needs_layout_passes=False),
    )
    def sc_kernel(x_hbm, out_hbm, row_v, hist_v, hist2_v):
        wid = lax.axis_index("s") * 2 + lax.axis_index("c")
        _clear_hist(hist_v, 256 * L)
        for rr in range(rows_per_w):
            row = wid * rows_per_w + rr
            pltpu.sync_copy(x_hbm.at[row], row_v)
            _row_select(row_v, hist_v, hist2_v, nvec)
            pltpu.sync_copy(row_v, out_hbm.at[row])

    return sc_kernel(x)


# speedup vs baseline: 2.7343x; 2.7343x over previous
import functools

import jax
import jax.numpy as jnp
from jax import lax
from jax.experimental import pallas as pl
from jax.experimental.pallas import tpu as pltpu
from jax.experimental.pallas import tpu_sc as plsc

K = 256
L = 16

_ONES = None


def _clear_hist(hist_ref, nwords):
    @plsc.parallel_loop(0, nwords // L, unroll=8)
    def body(i):
        hist_ref[pl.ds(i * L, L)] = jnp.zeros((L,), jnp.int32)


def _merge_hist(hist_ref, hist2_ref):
    zeros = jnp.zeros((L,), jnp.int32)

    @plsc.parallel_loop(0, 256 // L)
    def body(jb):
        acc = zeros
        for lane in range(L):
            off = lane * 256 + jb * L
            acc = acc + hist_ref[pl.ds(off, L)]
            hist_ref[pl.ds(off, L)] = zeros
        hist2_ref[pl.ds(jb * L, L)] = acc


def _scan_level(hist_ref, r):
    lanes = lax.iota(jnp.int32, L)

    def body(jj, carry):
        suffix, best_b, cge_b, h_b = carry
        j = 15 - jj
        v = hist_ref[pl.ds(j * L, L)]
        rv = lax.rev(v, (0,))
        cs = plsc.cumsum(rv) + suffix
        m = cs >= r
        npos = jnp.max(plsc.all_reduce_population_count(m))
        l = jnp.max(plsc.all_reduce_ffs(m))
        sel = lanes == l
        cs_l = jnp.max(jnp.where(sel, cs, 0))
        rv_l = jnp.max(jnp.where(sel, rv, 0))
        b = j * L + (L - 1) - l
        upd = jnp.logical_and(npos > 0, best_b < 0)
        best_b = jnp.where(upd, b, best_b)
        cge_b = jnp.where(upd, cs_l, cge_b)
        h_b = jnp.where(upd, rv_l, h_b)
        suffix = suffix + jnp.sum(v)
        return suffix, best_b, cge_b, h_b

    z32 = jnp.int32(0)
    _, best_b, cge_b, h_b = lax.fori_loop(
        0, 256 // L, body, (z32, jnp.int32(-1), z32, z32)
    )
    return best_b, r - (cge_b - h_b)


def _row_select(row_ref, hist_ref, hist2_ref, nvec):
    ones = jnp.ones((L,), jnp.int32)
    lane_off = lax.iota(jnp.int32, L) * 256

    @plsc.parallel_loop(0, nvec, unroll=8)
    def l0(i):
        v = row_ref[pl.ds(i * L, L)]
        z = jnp.maximum(lax.bitcast_convert_type(v, jnp.int32), 0)
        row_ref[pl.ds(i * L, L)] = lax.bitcast_convert_type(z, jnp.float32)
        idx = lane_off + lax.shift_right_logical(z, 24)
        plsc.addupdate_scatter(hist_ref, [idx], ones)
    _merge_hist(hist_ref, hist2_ref)
    b, r = _scan_level(hist2_ref, jnp.int32(K))
    prefix = b << 24

    for shift in (16, 8, 0):
        ph = lax.shift_right_logical(prefix, shift + 8)

        @plsc.parallel_loop(0, nvec, unroll=8)
        def lj(i, shift=shift, ph=ph):
            z = lax.bitcast_convert_type(row_ref[pl.ds(i * L, L)], jnp.int32)
            m = lax.shift_right_logical(z, shift + 8) == ph
            idx = lane_off + (lax.shift_right_logical(z, shift) & 255)
            plsc.addupdate_scatter(hist_ref, [idx], ones, mask=m)
        _merge_hist(hist_ref, hist2_ref)
        b, r = _scan_level(hist2_ref, r)
        prefix = prefix | (b << shift)

    @plsc.parallel_loop(0, nvec, unroll=8)
    def out_body(i):
        zf = row_ref[pl.ds(i * L, L)]
        z = lax.bitcast_convert_type(zf, jnp.int32)
        row_ref[pl.ds(i * L, L)] = jnp.where(z >= prefix, zf, 0.0)


def kernel(x):
    m, n = x.shape
    nw = 32
    rows_per_w = m // nw
    nvec = n // L
    mesh = plsc.VectorSubcoreMesh(
        core_axis_name="c", subcore_axis_name="s", num_cores=2, num_subcores=16
    )

    @functools.partial(
        pl.kernel,
        out_type=jax.ShapeDtypeStruct((m, n), jnp.float32),
        mesh=mesh,
        scratch_types=[
            pltpu.VMEM((n,), jnp.float32),
            pltpu.VMEM((256 * L,), jnp.int32),
            pltpu.VMEM((256,), jnp.int32),
        ],
        compiler_params=pltpu.CompilerParams(needs_layout_passes=False),
    )
    def sc_kernel(x_hbm, out_hbm, row_v, hist_v, hist2_v):
        wid = lax.axis_index("s") * 2 + lax.axis_index("c")
        _clear_hist(hist_v, 256 * L)
        for rr in range(rows_per_w):
            row = wid * rows_per_w + rr
            pltpu.sync_copy(x_hbm.at[row], row_v)
            _row_select(row_v, hist_v, hist2_v, nvec)
            pltpu.sync_copy(row_v, out_hbm.at[row])

    return sc_kernel(x)
